# Initial kernel scaffold; baseline (speedup 1.0000x reference)
import jax
import jax.numpy as jnp
from jax.experimental import pallas as pl


def kernel(inputs, entire_inputs, gl_w, gl_b, gl_fc_w, gl_fc_b, gl_out_w, gl_out_b, enc_Wru0, enc_Wru1, enc_bru, enc_Wc0, enc_Wc1, enc_bc, dec_Wru0, dec_Wru1, dec_bru, dec_Wc0, dec_Wc1, dec_bc):
    raise NotImplementedError("write your pallas kernel here")


# speedup vs baseline: 49.9674x; 49.9674x over previous
import functools

import jax
import jax.numpy as jnp
from jax.experimental import pallas as pl
from jax.experimental.pallas import tpu as pltpu

N = 256
TAU = 0.5
F_IN = 2
OUT_DIM = 1
H = 32
T_ENC = 4
T_DEC = 4
GLH = 64
B = 4

_C = 16


def _fused(x_ref, ent_ref, glw_ref, glb_ref, glfc_ref, glfcb_ref,
           gow_ref, gob_ref, g0_ref, g1_ref,
           eWru0_ref, eWru1_ref, ebru_ref, eWc0_ref, eWc1_ref, ebc_ref,
           dWru0_ref, dWru1_ref, dbru_ref, dWc0_ref, dWc1_ref, dbc_ref,
           out_ref, emb_ref, adjT_ref):
    f32 = jnp.float32
    dot = functools.partial(jnp.dot, preferred_element_type=f32)
    hdot = functools.partial(jnp.dot, preferred_element_type=f32,
                             precision=jax.lax.Precision.HIGHEST)

    emb = jnp.maximum(dot(ent_ref[...], glw_ref[...]) + glb_ref[...], 0.0)
    emb_ref[...] = emb
    glfc = glfc_ref[...]
    glfcb = glfcb_ref[...]
    gow = gow_ref[...]
    gob = gob_ref[...]
    src_id = jax.lax.broadcasted_iota(jnp.int32, (_C, N), 1)
    dst_id0 = jax.lax.broadcasted_iota(jnp.int32, (_C, N), 0)

    def mask_chunk(c, carry):
        embc = emb_ref[pl.ds(c * _C, _C), :]
        left = jnp.broadcast_to(emb[None, :, :], (_C, N, GLH)).reshape(_C * N, GLH)
        right = jnp.broadcast_to(embc[:, None, :], (_C, N, GLH)).reshape(_C * N, GLH)
        pair = jnp.concatenate([left, right], axis=1)
        hp = jnp.maximum(dot(pair, glfc) + glfcb, 0.0)
        z = (dot(hp, gow) + gob).reshape(_C, N, 2)
        v0 = (z[:, :, 0] + g0_ref[pl.ds(c * _C, _C), :]) / TAU
        v1 = (z[:, :, 1] + g1_ref[pl.ds(c * _C, _C), :]) / TAU
        keep = (v0 >= v1) & (src_id != (dst_id0 + c * _C))
        adjT_ref[pl.ds(c * _C, _C), :] = jnp.where(keep, 1.0, 0.0).astype(f32)
        return carry

    jax.lax.fori_loop(0, N // _C, mask_chunk, 0)
    adjT = adjT_ref[...]

    def agg(v):
        return jnp.stack([hdot(adjT, v[k]) for k in range(B)], axis=0)

    def cell(x, h, Wru0, Wru1, bru, Wc0, Wc1, bc):
        cat = jnp.concatenate([x, h], axis=-1)
        cin = cat.shape[-1]
        acat = agg(cat)
        cat2d = cat.reshape(B * N, cin)
        acat2d = acat.reshape(B * N, cin)
        ru = jax.nn.sigmoid(dot(cat2d, Wru0) + dot(acat2d, Wru1) + bru)
        r, u = ru[:, :H], ru[:, H:]
        h2 = h.reshape(B * N, H)
        rh = r * h2
        cat2 = jnp.concatenate([x.reshape(B * N, -1), rh], axis=-1)
        acat2 = agg(cat2.reshape(B, N, cin)).reshape(B * N, cin)
        cand = jnp.tanh(dot(cat2, Wc0) + dot(acat2, Wc1) + bc)
        return (u * h2 + (1.0 - u) * cand).reshape(B, N, H)

    h = jnp.zeros((B, N, H), dtype=f32)
    for t in range(T_ENC):
        h = cell(x_ref[t], h, eWru0_ref[...], eWru1_ref[...], ebru_ref[...],
                 eWc0_ref[...], eWc1_ref[...], ebc_ref[...])

    go = jnp.zeros((B, N, OUT_DIM), dtype=f32)
    for t in range(T_DEC):
        h = cell(go, h, dWru0_ref[...], dWru1_ref[...], dbru_ref[...],
                 dWc0_ref[...], dWc1_ref[...], dbc_ref[...])
        out_ref[t] = h


def _gumbel_planes():
    E = N * (N - 1)
    g = jax.random.gumbel(jax.random.key(42), (E, 2), dtype=jnp.float32)
    rows = jnp.arange(N)[:, None]
    cols = jnp.arange(N)[None, :]
    take = jnp.clip(jnp.where(cols < rows, cols, cols - 1), 0, N - 2)
    planes = []
    for c in range(2):
        gc = g[:, c].reshape(N, N - 1)
        dense = jnp.take_along_axis(gc, take, axis=1)
        planes.append(dense.T)
    return planes


def kernel(inputs, entire_inputs, gl_w, gl_b, gl_fc_w, gl_fc_b, gl_out_w,
           gl_out_b, enc_Wru0, enc_Wru1, enc_bru, enc_Wc0, enc_Wc1, enc_bc,
           dec_Wru0, dec_Wru1, dec_bru, dec_Wc0, dec_Wc1, dec_bc):
    f32 = jnp.float32
    g0, g1 = _gumbel_planes()
    args = (
        inputs,
        entire_inputs,
        gl_w,
        gl_b.reshape(1, GLH),
        gl_fc_w,
        gl_fc_b.reshape(1, GLH),
        gl_out_w,
        gl_out_b.reshape(1, 2),
        g0, g1,
        enc_Wru0, enc_Wru1, enc_bru.reshape(1, 2 * H),
        enc_Wc0, enc_Wc1, enc_bc.reshape(1, H),
        dec_Wru0, dec_Wru1, dec_bru.reshape(1, 2 * H),
        dec_Wc0, dec_Wc1, dec_bc.reshape(1, H),
    )
    out = pl.pallas_call(
        _fused,
        out_shape=jax.ShapeDtypeStruct((T_DEC, B, N, H), f32),
        scratch_shapes=[
            pltpu.VMEM((N, GLH), f32),
            pltpu.VMEM((N, N), f32),
        ],
    )(*args)
    return out


# speedup vs baseline: 73.7239x; 1.4754x over previous
import functools

import jax
import jax.numpy as jnp
from jax.experimental import pallas as pl
from jax.experimental.pallas import tpu as pltpu

N = 256
TAU = 0.5
F_IN = 2
OUT_DIM = 1
H = 32
T_ENC = 4
T_DEC = 4
GLH = 64
B = 4

_C = 16


def _fused(x_ref, ent_ref, glw_ref, glb_ref, glfc_ref, glfcb_ref,
           gow_ref, gob_ref, g0_ref, g1_ref,
           eWru0_ref, eWru1_ref, ebru_ref, eWc0_ref, eWc1_ref, ebc_ref,
           dWru0_ref, dWru1_ref, dbru_ref, dWc0_ref, dWc1_ref, dbc_ref,
           out_ref, emb_ref, adjT_ref, pair_ref):
    f32 = jnp.float32
    dot = functools.partial(jnp.dot, preferred_element_type=f32)
    hdot = functools.partial(jnp.dot, preferred_element_type=f32,
                             precision=jax.lax.Precision.HIGHEST)

    emb = jnp.maximum(dot(ent_ref[...], glw_ref[...]) + glb_ref[...], 0.0)
    emb_ref[...] = emb
    glfc = glfc_ref[...]
    glfcb = glfcb_ref[...]
    gow = gow_ref[...]
    gob = gob_ref[...]
    src_id = jax.lax.broadcasted_iota(jnp.int32, (_C, N), 1)
    dst_id0 = jax.lax.broadcasted_iota(jnp.int32, (_C, N), 0)

    pair_ref[:, :GLH] = jnp.broadcast_to(
        emb[None, :, :], (_C, N, GLH)).reshape(_C * N, GLH)

    def mask_chunk(c, carry):
        embc = emb_ref[pl.ds(c * _C, _C), :]
        pair_ref[:, GLH:] = jnp.broadcast_to(
            embc[:, None, :], (_C, N, GLH)).reshape(_C * N, GLH)
        hp = jnp.maximum(dot(pair_ref[...], glfc) + glfcb, 0.0)
        z = (dot(hp, gow) + gob).reshape(_C, N, 2)
        v0 = (z[:, :, 0] + g0_ref[pl.ds(c * _C, _C), :]) / TAU
        v1 = (z[:, :, 1] + g1_ref[pl.ds(c * _C, _C), :]) / TAU
        keep = (v0 >= v1) & (src_id != (dst_id0 + c * _C))
        adjT_ref[pl.ds(c * _C, _C), :] = jnp.where(keep, 1.0, 0.0).astype(f32)
        return carry

    jax.lax.fori_loop(0, N // _C, mask_chunk, 0)
    adjT = adjT_ref[...]

    def agg(v):
        return jnp.stack([hdot(adjT, v[k]) for k in range(B)], axis=0)

    def cell(x, h, Wru0, Wru1, bru, Wc0, Wc1, bc):
        cat = jnp.concatenate([x, h], axis=-1)
        cin = cat.shape[-1]
        acat = agg(cat)
        cat2d = cat.reshape(B * N, cin)
        acat2d = acat.reshape(B * N, cin)
        ru = jax.nn.sigmoid(dot(cat2d, Wru0) + dot(acat2d, Wru1) + bru)
        r, u = ru[:, :H], ru[:, H:]
        h2 = h.reshape(B * N, H)
        rh = r * h2
        cat2 = jnp.concatenate([x.reshape(B * N, -1), rh], axis=-1)
        acat2 = agg(cat2.reshape(B, N, cin)).reshape(B * N, cin)
        cand = jnp.tanh(dot(cat2, Wc0) + dot(acat2, Wc1) + bc)
        return (u * h2 + (1.0 - u) * cand).reshape(B, N, H)

    h = jnp.zeros((B, N, H), dtype=f32)
    for t in range(T_ENC):
        h = cell(x_ref[t], h, eWru0_ref[...], eWru1_ref[...], ebru_ref[...],
                 eWc0_ref[...], eWc1_ref[...], ebc_ref[...])

    go = jnp.zeros((B, N, OUT_DIM), dtype=f32)
    for t in range(T_DEC):
        h = cell(go, h, dWru0_ref[...], dWru1_ref[...], dbru_ref[...],
                 dWc0_ref[...], dWc1_ref[...], dbc_ref[...])
        out_ref[t] = h


@functools.lru_cache(maxsize=1)
def _gumbel_planes():
    with jax.ensure_compile_time_eval():
        E = N * (N - 1)
        g = jax.random.gumbel(jax.random.key(42), (E, 2), dtype=jnp.float32)
        rows = jnp.arange(N)[:, None]
        cols = jnp.arange(N)[None, :]
        take = jnp.clip(jnp.where(cols < rows, cols, cols - 1), 0, N - 2)
        planes = []
        for c in range(2):
            gc = g[:, c].reshape(N, N - 1)
            dense = jnp.take_along_axis(gc, take, axis=1)
            planes.append(dense.T)
        return tuple(planes)


def kernel(inputs, entire_inputs, gl_w, gl_b, gl_fc_w, gl_fc_b, gl_out_w,
           gl_out_b, enc_Wru0, enc_Wru1, enc_bru, enc_Wc0, enc_Wc1, enc_bc,
           dec_Wru0, dec_Wru1, dec_bru, dec_Wc0, dec_Wc1, dec_bc):
    f32 = jnp.float32
    g0, g1 = _gumbel_planes()
    args = (
        inputs,
        entire_inputs,
        gl_w,
        gl_b.reshape(1, GLH),
        gl_fc_w,
        gl_fc_b.reshape(1, GLH),
        gl_out_w,
        gl_out_b.reshape(1, 2),
        g0, g1,
        enc_Wru0, enc_Wru1, enc_bru.reshape(1, 2 * H),
        enc_Wc0, enc_Wc1, enc_bc.reshape(1, H),
        dec_Wru0, dec_Wru1, dec_bru.reshape(1, 2 * H),
        dec_Wc0, dec_Wc1, dec_bc.reshape(1, H),
    )
    out = pl.pallas_call(
        _fused,
        out_shape=jax.ShapeDtypeStruct((T_DEC, B, N, H), f32),
        scratch_shapes=[
            pltpu.VMEM((N, GLH), f32),
            pltpu.VMEM((N, N), f32),
            pltpu.VMEM((_C * N, 2 * GLH), f32),
        ],
    )(*args)
    return out


# speedup vs baseline: 73.9061x; 1.0025x over previous
import functools

import jax
import jax.numpy as jnp
from jax.experimental import pallas as pl
from jax.experimental.pallas import tpu as pltpu

N = 256
TAU = 0.5
F_IN = 2
OUT_DIM = 1
H = 32
T_ENC = 4
T_DEC = 4
GLH = 64
B = 4

_C = 32


def _fused(x_ref, ent_ref, glw_ref, glb_ref, glfc_ref, glfcb_ref,
           gow_ref, gob_ref, g0_ref, g1_ref,
           eWru0_ref, eWru1_ref, ebru_ref, eWc0_ref, eWc1_ref, ebc_ref,
           dWru0_ref, dWru1_ref, dbru_ref, dWc0_ref, dWc1_ref, dbc_ref,
           out_ref, emb_ref, adjT_ref, pair_ref):
    f32 = jnp.float32
    dot = functools.partial(jnp.dot, preferred_element_type=f32)
    hdot = functools.partial(jnp.dot, preferred_element_type=f32,
                             precision=jax.lax.Precision.HIGHEST)

    emb = jnp.maximum(dot(ent_ref[...], glw_ref[...]) + glb_ref[...], 0.0)
    emb_ref[...] = emb
    glfc = glfc_ref[...]
    glfcb = glfcb_ref[...]
    gow = gow_ref[...]
    gob = gob_ref[...]
    src_id = jax.lax.broadcasted_iota(jnp.int32, (_C, N), 1)
    dst_id0 = jax.lax.broadcasted_iota(jnp.int32, (_C, N), 0)

    pair_ref[:, :GLH] = jnp.broadcast_to(
        emb[None, :, :], (_C, N, GLH)).reshape(_C * N, GLH)

    def mask_chunk(c, carry):
        embc = emb_ref[pl.ds(c * _C, _C), :]
        pair_ref[:, GLH:] = jnp.broadcast_to(
            embc[:, None, :], (_C, N, GLH)).reshape(_C * N, GLH)
        hp = jnp.maximum(dot(pair_ref[...], glfc) + glfcb, 0.0)
        z = (dot(hp, gow) + gob).reshape(_C, N, 2)
        v0 = (z[:, :, 0] + g0_ref[pl.ds(c * _C, _C), :]) / TAU
        v1 = (z[:, :, 1] + g1_ref[pl.ds(c * _C, _C), :]) / TAU
        keep = (v0 >= v1) & (src_id != (dst_id0 + c * _C))
        adjT_ref[pl.ds(c * _C, _C), :] = jnp.where(keep, 1.0, 0.0).astype(f32)
        return carry

    jax.lax.fori_loop(0, N // _C, mask_chunk, 0)
    adjT = adjT_ref[...]

    def agg(v):
        return jnp.stack([hdot(adjT, v[k]) for k in range(B)], axis=0)

    def cell(x, h, Wru0, Wru1, bru, Wc0, Wc1, bc):
        cat = jnp.concatenate([x, h], axis=-1)
        cin = cat.shape[-1]
        acat = agg(cat)
        cat2d = cat.reshape(B * N, cin)
        acat2d = acat.reshape(B * N, cin)
        ru = jax.nn.sigmoid(dot(cat2d, Wru0) + dot(acat2d, Wru1) + bru)
        r, u = ru[:, :H], ru[:, H:]
        h2 = h.reshape(B * N, H)
        rh = r * h2
        cat2 = jnp.concatenate([x.reshape(B * N, -1), rh], axis=-1)
        acat2 = agg(cat2.reshape(B, N, cin)).reshape(B * N, cin)
        cand = jnp.tanh(dot(cat2, Wc0) + dot(acat2, Wc1) + bc)
        return (u * h2 + (1.0 - u) * cand).reshape(B, N, H)

    h = jnp.zeros((B, N, H), dtype=f32)
    for t in range(T_ENC):
        h = cell(x_ref[t], h, eWru0_ref[...], eWru1_ref[...], ebru_ref[...],
                 eWc0_ref[...], eWc1_ref[...], ebc_ref[...])

    go = jnp.zeros((B, N, OUT_DIM), dtype=f32)
    for t in range(T_DEC):
        h = cell(go, h, dWru0_ref[...], dWru1_ref[...], dbru_ref[...],
                 dWc0_ref[...], dWc1_ref[...], dbc_ref[...])
        out_ref[t] = h


@functools.lru_cache(maxsize=1)
def _gumbel_planes():
    with jax.ensure_compile_time_eval():
        E = N * (N - 1)
        g = jax.random.gumbel(jax.random.key(42), (E, 2), dtype=jnp.float32)
        rows = jnp.arange(N)[:, None]
        cols = jnp.arange(N)[None, :]
        take = jnp.clip(jnp.where(cols < rows, cols, cols - 1), 0, N - 2)
        planes = []
        for c in range(2):
            gc = g[:, c].reshape(N, N - 1)
            dense = jnp.take_along_axis(gc, take, axis=1)
            planes.append(dense.T)
        return tuple(planes)


def kernel(inputs, entire_inputs, gl_w, gl_b, gl_fc_w, gl_fc_b, gl_out_w,
           gl_out_b, enc_Wru0, enc_Wru1, enc_bru, enc_Wc0, enc_Wc1, enc_bc,
           dec_Wru0, dec_Wru1, dec_bru, dec_Wc0, dec_Wc1, dec_bc):
    f32 = jnp.float32
    g0, g1 = _gumbel_planes()
    args = (
        inputs,
        entire_inputs,
        gl_w,
        gl_b.reshape(1, GLH),
        gl_fc_w,
        gl_fc_b.reshape(1, GLH),
        gl_out_w,
        gl_out_b.reshape(1, 2),
        g0, g1,
        enc_Wru0, enc_Wru1, enc_bru.reshape(1, 2 * H),
        enc_Wc0, enc_Wc1, enc_bc.reshape(1, H),
        dec_Wru0, dec_Wru1, dec_bru.reshape(1, 2 * H),
        dec_Wc0, dec_Wc1, dec_bc.reshape(1, H),
    )
    out = pl.pallas_call(
        _fused,
        out_shape=jax.ShapeDtypeStruct((T_DEC, B, N, H), f32),
        scratch_shapes=[
            pltpu.VMEM((N, GLH), f32),
            pltpu.VMEM((N, N), f32),
            pltpu.VMEM((_C * N, 2 * GLH), f32),
        ],
    )(*args)
    return out


# speedup vs baseline: 74.0814x; 1.0024x over previous
import functools

import jax
import jax.numpy as jnp
from jax.experimental import pallas as pl
from jax.experimental.pallas import tpu as pltpu

N = 256
TAU = 0.5
F_IN = 2
OUT_DIM = 1
H = 32
T_ENC = 4
T_DEC = 4
GLH = 64
B = 4

_C = 32


def _fused(x_ref, ent_ref, glw_ref, glb_ref, glfc_ref, glfcb_ref,
           gow_ref, gob_ref, g0_ref, g1_ref,
           eWru0_ref, eWru1_ref, ebru_ref, eWc0_ref, eWc1_ref, ebc_ref,
           dWru0_ref, dWru1_ref, dbru_ref, dWc0_ref, dWc1_ref, dbc_ref,
           out_ref, emb_ref, adjT_ref, pair_ref):
    f32 = jnp.float32
    dot = functools.partial(jnp.dot, preferred_element_type=f32)
    hdot = functools.partial(jnp.dot, preferred_element_type=f32,
                             precision=jax.lax.Precision.HIGHEST)

    emb = jnp.maximum(dot(ent_ref[...], glw_ref[...]) + glb_ref[...], 0.0)
    emb_ref[...] = emb
    glfc = glfc_ref[...]
    glfcb = glfcb_ref[...]
    gow = gow_ref[...]
    gob = gob_ref[...]
    src_id = jax.lax.broadcasted_iota(jnp.int32, (_C, N), 1)
    dst_id0 = jax.lax.broadcasted_iota(jnp.int32, (_C, N), 0)

    bf = jnp.bfloat16
    emb_b = emb.astype(bf)
    glfc_b = glfc.astype(bf)
    gow_b = gow.astype(bf)

    pair_ref[:, :GLH] = jnp.broadcast_to(
        emb_b[None, :, :], (_C, N, GLH)).reshape(_C * N, GLH)

    def mask_chunk(c, carry):
        embc = emb_ref[pl.ds(c * _C, _C), :].astype(bf)
        pair_ref[:, GLH:] = jnp.broadcast_to(
            embc[:, None, :], (_C, N, GLH)).reshape(_C * N, GLH)
        hp = jnp.maximum(dot(pair_ref[...], glfc_b) + glfcb, 0.0)
        z = (dot(hp.astype(bf), gow_b) + gob).reshape(_C, N, 2)
        v0 = (z[:, :, 0] + g0_ref[pl.ds(c * _C, _C), :]) / TAU
        v1 = (z[:, :, 1] + g1_ref[pl.ds(c * _C, _C), :]) / TAU
        keep = (v0 >= v1) & (src_id != (dst_id0 + c * _C))
        adjT_ref[pl.ds(c * _C, _C), :] = jnp.where(keep, 1.0, 0.0).astype(f32)
        return carry

    jax.lax.fori_loop(0, N // _C, mask_chunk, 0)
    adjT = adjT_ref[...]

    def agg(v):
        return jnp.stack([hdot(adjT, v[k]) for k in range(B)], axis=0)

    def cell(x, h, Wru0, Wru1, bru, Wc0, Wc1, bc):
        cat = jnp.concatenate([x, h], axis=-1)
        cin = cat.shape[-1]
        acat = agg(cat)
        cat2d = cat.reshape(B * N, cin)
        acat2d = acat.reshape(B * N, cin)
        ru = jax.nn.sigmoid(dot(cat2d, Wru0) + dot(acat2d, Wru1) + bru)
        r, u = ru[:, :H], ru[:, H:]
        h2 = h.reshape(B * N, H)
        rh = r * h2
        cat2 = jnp.concatenate([x.reshape(B * N, -1), rh], axis=-1)
        acat2 = agg(cat2.reshape(B, N, cin)).reshape(B * N, cin)
        cand = jnp.tanh(dot(cat2, Wc0) + dot(acat2, Wc1) + bc)
        return (u * h2 + (1.0 - u) * cand).reshape(B, N, H)

    h = jnp.zeros((B, N, H), dtype=f32)
    for t in range(T_ENC):
        h = cell(x_ref[t], h, eWru0_ref[...], eWru1_ref[...], ebru_ref[...],
                 eWc0_ref[...], eWc1_ref[...], ebc_ref[...])

    go = jnp.zeros((B, N, OUT_DIM), dtype=f32)
    for t in range(T_DEC):
        h = cell(go, h, dWru0_ref[...], dWru1_ref[...], dbru_ref[...],
                 dWc0_ref[...], dWc1_ref[...], dbc_ref[...])
        out_ref[t] = h


@functools.lru_cache(maxsize=1)
def _gumbel_planes():
    with jax.ensure_compile_time_eval():
        E = N * (N - 1)
        g = jax.random.gumbel(jax.random.key(42), (E, 2), dtype=jnp.float32)
        rows = jnp.arange(N)[:, None]
        cols = jnp.arange(N)[None, :]
        take = jnp.clip(jnp.where(cols < rows, cols, cols - 1), 0, N - 2)
        planes = []
        for c in range(2):
            gc = g[:, c].reshape(N, N - 1)
            dense = jnp.take_along_axis(gc, take, axis=1)
            planes.append(dense.T)
        return tuple(planes)


def kernel(inputs, entire_inputs, gl_w, gl_b, gl_fc_w, gl_fc_b, gl_out_w,
           gl_out_b, enc_Wru0, enc_Wru1, enc_bru, enc_Wc0, enc_Wc1, enc_bc,
           dec_Wru0, dec_Wru1, dec_bru, dec_Wc0, dec_Wc1, dec_bc):
    f32 = jnp.float32
    g0, g1 = _gumbel_planes()
    args = (
        inputs,
        entire_inputs,
        gl_w,
        gl_b.reshape(1, GLH),
        gl_fc_w,
        gl_fc_b.reshape(1, GLH),
        gl_out_w,
        gl_out_b.reshape(1, 2),
        g0, g1,
        enc_Wru0, enc_Wru1, enc_bru.reshape(1, 2 * H),
        enc_Wc0, enc_Wc1, enc_bc.reshape(1, H),
        dec_Wru0, dec_Wru1, dec_bru.reshape(1, 2 * H),
        dec_Wc0, dec_Wc1, dec_bc.reshape(1, H),
    )
    out = pl.pallas_call(
        _fused,
        out_shape=jax.ShapeDtypeStruct((T_DEC, B, N, H), f32),
        scratch_shapes=[
            pltpu.VMEM((N, GLH), f32),
            pltpu.VMEM((N, N), f32),
            pltpu.VMEM((_C * N, 2 * GLH), jnp.bfloat16),
        ],
    )(*args)
    return out
